# baseline (speedup 1.0000x reference)
import jax
import jax.numpy as jnp
from jax import lax
from jax.experimental import pallas as pl
from jax.experimental.pallas import tpu as pltpu
from jax.experimental.pallas import tpu_sc as plsc

_T = 2048
_D = 1024
_DW = _D // 2
_E = 8
_C = 512
_DFF = 2048
_SLOTS = _E * _C
_TRASH = _SLOTS
_DISP_ROWS = _SLOTS + 8

_NC, _NS = 2, 16
_NW = _NC * _NS
_TPW = _T // _NW
_CH = 32
_NCH = _TPW // _CH

_FFC = 1024
_NFF = _DFF // _FFC


def _rne_bf16_bits(f):
    u = lax.bitcast_convert_type(f, jnp.uint32)
    rounded = u + 0x7FFF + ((u >> 16) & 1)
    return rounded >> 16


def _pack_f32(x):
    lo = _rne_bf16_bits(x[:, :_DW])
    hi = _rne_bf16_bits(x[:, _DW:])
    return lax.bitcast_convert_type(lo | (hi << 16), jnp.int32)


def _unpack_f32(p):
    u = lax.bitcast_convert_type(p, jnp.uint32)
    lo = lax.bitcast_convert_type(u << 16, jnp.float32)
    hi = lax.bitcast_convert_type(u & jnp.uint32(0xFFFF0000), jnp.float32)
    return jnp.concatenate([lo, hi], axis=1)


def _excl_cumsum_tokens(x):
    rows = lax.broadcasted_iota(jnp.int32, x.shape, 0)
    y = x
    k = 1
    while k < x.shape[0]:
        shifted = pltpu.roll(y, k, 0)
        y = y + jnp.where(rows >= k, shifted, 0.0)
        k *= 2
    return y - x


def _route_body(tok_ref, gw_ref, dst1_ref, dst2_ref, cdst1_ref, cdst2_ref,
                gm1_ref, gm2_ref, tokp_ref):
    x = tok_ref[...]
    tokp_ref[...] = _pack_f32(x)
    gw = gw_ref[...]
    logits = lax.dot_general(x, gw, (((1,), (1,)), ((), ())),
                             preferred_element_type=jnp.float32)
    mx = jnp.max(logits, axis=1, keepdims=True)
    ex = jnp.exp(logits - mx)
    probs = ex / jnp.sum(ex, axis=1, keepdims=True)

    idx = lax.broadcasted_iota(jnp.int32, (_T, _E), 1)
    p1 = jnp.max(probs, axis=1, keepdims=True)
    e1 = jnp.min(jnp.where(probs == p1, idx, _E), axis=1, keepdims=True)
    sel1 = idx == e1
    probs2 = jnp.where(sel1, -1.0, probs)
    p2 = jnp.max(probs2, axis=1, keepdims=True)
    e2 = jnp.min(jnp.where(probs2 == p2, idx, _E), axis=1, keepdims=True)
    sel2 = idx == e2

    denom = p1 + p2 + 1e-9
    g1 = p1 / denom
    g2 = p2 / denom

    m1 = sel1.astype(jnp.float32)
    m2 = sel2.astype(jnp.float32)
    c12 = _excl_cumsum_tokens(jnp.concatenate([m1, m2], axis=1))
    c1 = c12[:, :_E]
    tot1 = jnp.sum(m1, axis=0, keepdims=True)
    c2 = c12[:, _E:] + tot1
    pos1 = jnp.sum(c1 * m1, axis=1, keepdims=True)
    pos2 = jnp.sum(c2 * m2, axis=1, keepdims=True)
    v1 = pos1 < _C
    v2 = pos2 < _C
    slot1 = e1 * _C + pos1.astype(jnp.int32)
    slot2 = e2 * _C + pos2.astype(jnp.int32)

    dst1_ref[...] = jnp.where(v1, slot1, _TRASH)[:, 0]
    dst2_ref[...] = jnp.where(v2, slot2, _TRASH)[:, 0]
    cdst1_ref[...] = jnp.where(v1, slot1, 0)[:, 0]
    cdst2_ref[...] = jnp.where(v2, slot2, 0)[:, 0]
    gm1_ref[...] = jnp.where(v1, g1, -1.0)
    gm2_ref[...] = jnp.where(v2, g2, -1.0)


def _route(tokens, gate_w):
    i32 = jnp.int32
    outs = jax.ShapeDtypeStruct
    return pl.pallas_call(
        _route_body,
        out_shape=[outs((_T,), i32), outs((_T,), i32),
                   outs((_T,), i32), outs((_T,), i32),
                   outs((_T, 1), jnp.float32), outs((_T, 1), jnp.float32),
                   outs((_T, _DW), i32)],
    )(tokens, gate_w)


def _dispatch_body(tok_hbm, dst1_hbm, dst2_hbm, disp_hbm,
                   idx1_v, idx2_v, rows_v, sem):
    wid = lax.axis_index("s") * _NC + lax.axis_index("c")
    base = wid * _TPW
    pltpu.sync_copy(dst1_hbm.at[pl.ds(base, _TPW)], idx1_v)
    pltpu.sync_copy(dst2_hbm.at[pl.ds(base, _TPW)], idx2_v)
    pltpu.sync_copy(tok_hbm.at[pl.ds(base, _TPW)], rows_v)
    d1 = pltpu.async_copy(rows_v, disp_hbm.at[idx1_v], sem)
    d2 = pltpu.async_copy(rows_v, disp_hbm.at[idx2_v], sem)
    d1.wait()
    d2.wait()


def _dispatch(tokp, dst1, dst2):
    mesh = plsc.VectorSubcoreMesh(core_axis_name="c", subcore_axis_name="s")
    return pl.kernel(
        _dispatch_body,
        out_type=jax.ShapeDtypeStruct((_DISP_ROWS, _DW), jnp.int32),
        mesh=mesh,
        scratch_types=[
            pltpu.VMEM((_TPW,), jnp.int32),
            pltpu.VMEM((_TPW,), jnp.int32),
            pltpu.VMEM((_TPW, _DW), jnp.int32),
            pltpu.SemaphoreType.DMA,
        ],
    )(tokp, dst1, dst2)


def _ffn_body(xp_ref, w1_ref, w2_ref, o_ref, x16_ref, acc_ref):
    f = pl.program_id(1)

    @pl.when(f == 0)
    def _():
        x16_ref[...] = _unpack_f32(xp_ref[...]).astype(jnp.bfloat16)

    h = jnp.maximum(
        lax.dot_general(x16_ref[...], w1_ref[0].astype(jnp.bfloat16),
                        (((1,), (0,)), ((), ())),
                        preferred_element_type=jnp.float32), 0.0)
    y = lax.dot_general(h.astype(jnp.bfloat16), w2_ref[0].astype(jnp.bfloat16),
                        (((1,), (0,)), ((), ())),
                        preferred_element_type=jnp.float32)

    @pl.when(f == 0)
    def _():
        acc_ref[...] = y

    @pl.when(f != 0)
    def _():
        acc_ref[...] += y

    @pl.when(f == _NFF - 1)
    def _():
        o_ref[...] = _pack_f32(acc_ref[...])


def _ffn(disp, w1, w2):
    return pl.pallas_call(
        _ffn_body,
        grid=(_E, _NFF),
        in_specs=[
            pl.BlockSpec((_C, _DW), lambda e, f: (e, 0)),
            pl.BlockSpec((1, _D, _FFC), lambda e, f: (e, 0, f)),
            pl.BlockSpec((1, _FFC, _D), lambda e, f: (e, f, 0)),
        ],
        out_specs=pl.BlockSpec((_C, _DW), lambda e, f: (e, 0)),
        out_shape=jax.ShapeDtypeStruct((_SLOTS, _DW), jnp.int32),
        scratch_shapes=[pltpu.VMEM((_C, _D), jnp.bfloat16),
                        pltpu.VMEM((_C, _D), jnp.float32)],
        compiler_params=pltpu.CompilerParams(
            dimension_semantics=("parallel", "arbitrary")),
    )(disp, w1, w2)


def _gather_body(eo_hbm, cdst1_hbm, cdst2_hbm, rows1_hbm, rows2_hbm,
                 i1_v, i2_v, r1_v, r2_v, sem):
    wid = lax.axis_index("s") * _NC + lax.axis_index("c")
    base = wid * _TPW
    pltpu.sync_copy(cdst1_hbm.at[pl.ds(base, _TPW)], i1_v)
    pltpu.sync_copy(cdst2_hbm.at[pl.ds(base, _TPW)], i2_v)
    d1 = pltpu.async_copy(eo_hbm.at[i1_v], r1_v, sem)
    d2 = pltpu.async_copy(eo_hbm.at[i2_v], r2_v, sem)
    d1.wait()
    pltpu.sync_copy(r1_v, rows1_hbm.at[pl.ds(base, _TPW)])
    d2.wait()
    pltpu.sync_copy(r2_v, rows2_hbm.at[pl.ds(base, _TPW)])


def _gather(eo, cdst1, cdst2):
    mesh = plsc.VectorSubcoreMesh(core_axis_name="c", subcore_axis_name="s")
    sh = jax.ShapeDtypeStruct((_T, _DW), jnp.int32)
    return pl.kernel(
        _gather_body,
        out_type=[sh, sh],
        mesh=mesh,
        scratch_types=[
            pltpu.VMEM((_TPW,), jnp.int32),
            pltpu.VMEM((_TPW,), jnp.int32),
            pltpu.VMEM((_TPW, _DW), jnp.int32),
            pltpu.VMEM((_TPW, _DW), jnp.int32),
            pltpu.SemaphoreType.DMA,
        ],
    )(eo, cdst1, cdst2)


def _mix_body(r1_ref, r2_ref, gm1_ref, gm2_ref, o_ref):
    r1 = _unpack_f32(r1_ref[...])
    r2 = _unpack_f32(r2_ref[...])
    g1 = gm1_ref[...]
    g2 = gm2_ref[...]
    o_ref[...] = (jnp.where(g1 >= 0.0, g1 * r1, 0.0)
                  + jnp.where(g2 >= 0.0, g2 * r2, 0.0))


def _mix(rows1, rows2, gm1, gm2):
    return pl.pallas_call(
        _mix_body,
        out_shape=jax.ShapeDtypeStruct((_T, _D), jnp.float32),
    )(rows1, rows2, gm1, gm2)


def kernel(inputs, gate_w, w1, w2):
    tokens = inputs.reshape(_T, _D).astype(jnp.float32)
    dst1, dst2, cdst1, cdst2, gm1, gm2, tokp = _route(tokens, gate_w)
    disp = _dispatch(tokp, dst1, dst2)
    eo = _ffn(disp, w1, w2)
    rows1, rows2 = _gather(eo, cdst1, cdst2)
    out = _mix(rows1, rows2, gm1, gm2)
    return out.reshape(inputs.shape)


# speedup vs baseline: 1.1254x; 1.1254x over previous
import jax
import jax.numpy as jnp
from jax import lax
from jax.experimental import pallas as pl
from jax.experimental.pallas import tpu as pltpu
from jax.experimental.pallas import tpu_sc as plsc

_T = 2048
_D = 1024
_DW = _D // 2
_E = 8
_C = 512
_DFF = 2048
_SLOTS = _E * _C
_TRASH = _SLOTS
_DISP_ROWS = _SLOTS + 8

_NC, _NS = 2, 16
_NW = _NC * _NS
_TPW = _T // _NW
_CH = 32
_NCH = _TPW // _CH

_FFC = 2048
_NFF = _DFF // _FFC


def _rne_bf16_bits(f):
    u = lax.bitcast_convert_type(f, jnp.uint32)
    rounded = u + 0x7FFF + ((u >> 16) & 1)
    return rounded >> 16


def _pack_f32(x):
    lo = _rne_bf16_bits(x[:, :_DW])
    hi = _rne_bf16_bits(x[:, _DW:])
    return lax.bitcast_convert_type(lo | (hi << 16), jnp.int32)


def _unpack_f32(p):
    u = lax.bitcast_convert_type(p, jnp.uint32)
    lo = lax.bitcast_convert_type(u << 16, jnp.float32)
    hi = lax.bitcast_convert_type(u & jnp.uint32(0xFFFF0000), jnp.float32)
    return jnp.concatenate([lo, hi], axis=1)


def _excl_cumsum_lanes(x):
    cols = lax.broadcasted_iota(jnp.int32, x.shape, 1)
    y = x
    k = 1
    while k < x.shape[1]:
        shifted = pltpu.roll(y, k, 1)
        y = y + jnp.where(cols >= k, shifted, 0.0)
        k *= 2
    return y - x


def _route_body(tok_ref, gw_ref, dst1_ref, dst2_ref, cdst1_ref, cdst2_ref,
                gm1_ref, gm2_ref, tokp_ref):
    x = tok_ref[...]
    tokp_ref[...] = _pack_f32(x)
    gw = gw_ref[...]
    lt = lax.dot_general(gw, x, (((1,), (1,)), ((), ())),
                         preferred_element_type=jnp.float32)
    mx = jnp.max(lt, axis=0, keepdims=True)
    ex = jnp.exp(lt - mx)
    probs = ex / jnp.sum(ex, axis=0, keepdims=True)

    idx = lax.broadcasted_iota(jnp.int32, (_E, _T), 0)
    p1 = jnp.max(probs, axis=0, keepdims=True)
    e1 = jnp.min(jnp.where(probs == p1, idx, _E), axis=0, keepdims=True)
    sel1 = idx == e1
    probs2 = jnp.where(sel1, -1.0, probs)
    p2 = jnp.max(probs2, axis=0, keepdims=True)
    e2 = jnp.min(jnp.where(probs2 == p2, idx, _E), axis=0, keepdims=True)
    sel2 = idx == e2

    denom = p1 + p2 + 1e-9
    g1 = (p1 / denom)[0]
    g2 = (p2 / denom)[0]

    m1 = sel1.astype(jnp.float32)
    m2 = sel2.astype(jnp.float32)
    c12 = _excl_cumsum_lanes(jnp.concatenate([m1, m2], axis=0))
    c1 = c12[:_E]
    tot1 = jnp.sum(m1, axis=1, keepdims=True)
    c2 = c12[_E:] + tot1
    pos1 = jnp.sum(c1 * m1, axis=0)
    pos2 = jnp.sum(c2 * m2, axis=0)
    v1 = pos1 < _C
    v2 = pos2 < _C
    slot1 = e1[0] * _C + pos1.astype(jnp.int32)
    slot2 = e2[0] * _C + pos2.astype(jnp.int32)

    dst1_ref[...] = jnp.where(v1, slot1, _TRASH)
    dst2_ref[...] = jnp.where(v2, slot2, _TRASH)
    cdst1_ref[...] = jnp.where(v1, slot1, 0)
    cdst2_ref[...] = jnp.where(v2, slot2, 0)
    gm1_ref[...] = jnp.where(v1, g1, -1.0)
    gm2_ref[...] = jnp.where(v2, g2, -1.0)


def _route(tokens, gate_w):
    i32 = jnp.int32
    outs = jax.ShapeDtypeStruct
    return pl.pallas_call(
        _route_body,
        out_shape=[outs((_T,), i32), outs((_T,), i32),
                   outs((_T,), i32), outs((_T,), i32),
                   outs((_T,), jnp.float32), outs((_T,), jnp.float32),
                   outs((_T, _DW), i32)],
    )(tokens, gate_w)


def _dispatch_body(tok_hbm, dst1_hbm, dst2_hbm, disp_hbm,
                   idx1_v, idx2_v, rows_v, sem):
    wid = lax.axis_index("s") * _NC + lax.axis_index("c")
    base = wid * _TPW
    pltpu.sync_copy(dst1_hbm.at[pl.ds(base, _TPW)], idx1_v)
    pltpu.sync_copy(dst2_hbm.at[pl.ds(base, _TPW)], idx2_v)
    pltpu.sync_copy(tok_hbm.at[pl.ds(base, _TPW)], rows_v)
    d1 = pltpu.async_copy(rows_v, disp_hbm.at[idx1_v], sem)
    d2 = pltpu.async_copy(rows_v, disp_hbm.at[idx2_v], sem)
    d1.wait()
    d2.wait()


def _dispatch(tokp, dst1, dst2):
    mesh = plsc.VectorSubcoreMesh(core_axis_name="c", subcore_axis_name="s")
    return pl.kernel(
        _dispatch_body,
        out_type=jax.ShapeDtypeStruct((_DISP_ROWS, _DW), jnp.int32),
        mesh=mesh,
        scratch_types=[
            pltpu.VMEM((_TPW,), jnp.int32),
            pltpu.VMEM((_TPW,), jnp.int32),
            pltpu.VMEM((_TPW, _DW), jnp.int32),
            pltpu.SemaphoreType.DMA,
        ],
    )(tokp, dst1, dst2)


def _ffn_body(xp_ref, w1_ref, w2_ref, o_ref, x16_ref, acc_ref):
    f = pl.program_id(1)

    @pl.when(f == 0)
    def _():
        x16_ref[...] = _unpack_f32(xp_ref[...]).astype(jnp.bfloat16)

    h = jnp.maximum(
        lax.dot_general(x16_ref[...], w1_ref[0].astype(jnp.bfloat16),
                        (((1,), (0,)), ((), ())),
                        preferred_element_type=jnp.float32), 0.0)
    y = lax.dot_general(h.astype(jnp.bfloat16), w2_ref[0].astype(jnp.bfloat16),
                        (((1,), (0,)), ((), ())),
                        preferred_element_type=jnp.float32)

    @pl.when(f == 0)
    def _():
        acc_ref[...] = y

    @pl.when(f != 0)
    def _():
        acc_ref[...] += y

    @pl.when(f == _NFF - 1)
    def _():
        o_ref[...] = _pack_f32(acc_ref[...])


def _ffn(disp, w1, w2):
    return pl.pallas_call(
        _ffn_body,
        grid=(_E, _NFF),
        in_specs=[
            pl.BlockSpec((_C, _DW), lambda e, f: (e, 0)),
            pl.BlockSpec((1, _D, _FFC), lambda e, f: (e, 0, f)),
            pl.BlockSpec((1, _FFC, _D), lambda e, f: (e, f, 0)),
        ],
        out_specs=pl.BlockSpec((_C, _DW), lambda e, f: (e, 0)),
        out_shape=jax.ShapeDtypeStruct((_SLOTS, _DW), jnp.int32),
        scratch_shapes=[pltpu.VMEM((_C, _D), jnp.bfloat16),
                        pltpu.VMEM((_C, _D), jnp.float32)],
        compiler_params=pltpu.CompilerParams(
            dimension_semantics=("parallel", "arbitrary")),
    )(disp, w1, w2)


def _gather_body(eo_hbm, cdst1_hbm, cdst2_hbm, rows1_hbm, rows2_hbm,
                 i1_v, i2_v, r1_v, r2_v, sem):
    wid = lax.axis_index("s") * _NC + lax.axis_index("c")
    base = wid * _TPW
    pltpu.sync_copy(cdst1_hbm.at[pl.ds(base, _TPW)], i1_v)
    pltpu.sync_copy(cdst2_hbm.at[pl.ds(base, _TPW)], i2_v)
    d1 = pltpu.async_copy(eo_hbm.at[i1_v], r1_v, sem)
    d2 = pltpu.async_copy(eo_hbm.at[i2_v], r2_v, sem)
    d1.wait()
    pltpu.sync_copy(r1_v, rows1_hbm.at[pl.ds(base, _TPW)])
    d2.wait()
    pltpu.sync_copy(r2_v, rows2_hbm.at[pl.ds(base, _TPW)])


def _gather(eo, cdst1, cdst2):
    mesh = plsc.VectorSubcoreMesh(core_axis_name="c", subcore_axis_name="s")
    sh = jax.ShapeDtypeStruct((_T, _DW), jnp.int32)
    return pl.kernel(
        _gather_body,
        out_type=[sh, sh],
        mesh=mesh,
        scratch_types=[
            pltpu.VMEM((_TPW,), jnp.int32),
            pltpu.VMEM((_TPW,), jnp.int32),
            pltpu.VMEM((_TPW, _DW), jnp.int32),
            pltpu.VMEM((_TPW, _DW), jnp.int32),
            pltpu.SemaphoreType.DMA,
        ],
    )(eo, cdst1, cdst2)


def _mix_body(r1_ref, r2_ref, gm1_ref, gm2_ref, o_ref):
    r1 = _unpack_f32(r1_ref[...])
    r2 = _unpack_f32(r2_ref[...])
    g1 = gm1_ref[...].reshape(_T, 1)
    g2 = gm2_ref[...].reshape(_T, 1)
    o_ref[...] = (jnp.where(g1 >= 0.0, g1 * r1, 0.0)
                  + jnp.where(g2 >= 0.0, g2 * r2, 0.0))


def _mix(rows1, rows2, gm1, gm2):
    return pl.pallas_call(
        _mix_body,
        out_shape=jax.ShapeDtypeStruct((_T, _D), jnp.float32),
    )(rows1, rows2, gm1, gm2)


def kernel(inputs, gate_w, w1, w2):
    tokens = inputs.reshape(_T, _D).astype(jnp.float32)
    dst1, dst2, cdst1, cdst2, gm1, gm2, tokp = _route(tokens, gate_w)
    disp = _dispatch(tokp, dst1, dst2)
    eo = _ffn(disp, w1, w2)
    rows1, rows2 = _gather(eo, cdst1, cdst2)
    out = _mix(rows1, rows2, gm1, gm2)
    return out.reshape(inputs.shape)
